# Initial kernel scaffold; baseline (speedup 1.0000x reference)
import jax
import jax.numpy as jnp
from jax.experimental import pallas as pl


def kernel(x1, x2, W1, W2, bias1):
    raise NotImplementedError("write your pallas kernel here")


# speedup vs baseline: 114.4628x; 114.4628x over previous
import jax
import jax.numpy as jnp
from jax import lax
from jax.experimental import pallas as pl
from jax.experimental.pallas import tpu as pltpu
from jax.experimental.pallas import tpu_sc as plsc

_B = 16384
_F1 = 26
_F2 = 100
_V1 = 1000000
_V2 = 100001
_V2PAD = 100096

_NC = 2
_NS = 16
_L = 16
_NW = _NC * _NS
_SPW = _B // _NW
_SUB = 128
_NSUB = _SPW // _SUB
_JB = _SUB // _L


def _sc_body(x1f_hbm, x2f_hbm, w1_hbm, w2_hbm, bias_hbm, out_hbm,
             w2_v, x1s_v, x2s_v, idx1t_v, g1_v, outs_v, bias_v, sem):
    cid = lax.axis_index("c")
    sid = lax.axis_index("s")
    wid = sid * _NC + cid
    base = wid * _SPW

    pltpu.sync_copy(w2_hbm, w2_v)
    pltpu.sync_copy(bias_hbm, bias_v)

    iota = lax.iota(jnp.int32, _L)
    row1_j = [(iota + (_L * j)) * _F1 for j in range(_JB)]
    row2_j = [(iota + (_L * j)) * _F2 for j in range(_JB)]
    ones = jnp.ones((_L,), jnp.int32)

    def subblock(c, _):
        row0 = base + c * _SUB
        pltpu.sync_copy(x1f_hbm.at[pl.ds(row0 * _F1, _SUB * _F1)], x1s_v)
        pltpu.sync_copy(x2f_hbm.at[pl.ds(row0 * _F2, _SUB * _F2)], x2s_v)

        def tr_f(f, carry):
            off = ones * f
            for j in range(_JB):
                v = plsc.load_gather(x1s_v, [row1_j[j] + off])
                idx1t_v[pl.ds(f * _SUB + _L * j, _L)] = v
            return carry
        lax.fori_loop(0, _F1, tr_f, 0)

        def fire(f, carry):
            pltpu.async_copy(w1_hbm.at[idx1t_v.at[pl.ds(f * _SUB, _SUB)]],
                             g1_v.at[pl.ds(f * _SUB, _SUB)], sem)
            return carry
        lax.fori_loop(0, _F1, fire, 0)

        def f2_loop(f, accs):
            off = ones * f
            new = []
            for j in range(_JB):
                inds = plsc.load_gather(x2s_v, [row2_j[j] + off])
                vals = plsc.load_gather(w2_v, [inds])
                new.append(accs[j] + vals)
            return tuple(new)
        zeros = tuple(jnp.zeros((_L,), jnp.float32) for _ in range(_JB))
        accs = lax.fori_loop(0, _F2, f2_loop, zeros)

        def drain(f, carry):
            pltpu.make_async_copy(
                w1_hbm.at[idx1t_v.at[pl.ds(f * _SUB, _SUB)]],
                g1_v.at[pl.ds(f * _SUB, _SUB)], sem).wait()
            return carry
        lax.fori_loop(0, _F1, drain, 0)

        def f1_loop(f, accs_in):
            new = []
            for j in range(_JB):
                new.append(accs_in[j] + g1_v[pl.ds(f * _SUB + _L * j, _L)])
            return tuple(new)
        accs = lax.fori_loop(0, _F1, f1_loop, accs)

        bias = bias_v[...]
        for j in range(_JB):
            s = accs[j] + bias
            outs_v[pl.ds(c * _SUB + _L * j, _L)] = 1.0 / (1.0 + jnp.exp(-s))
        return 0

    lax.fori_loop(0, _NSUB, subblock, 0)
    pltpu.sync_copy(outs_v, out_hbm.at[pl.ds(base, _SPW)])


@jax.jit
def _run(x1f, x2f, w1, w2, bias16):
    mesh = plsc.VectorSubcoreMesh(
        core_axis_name="c", subcore_axis_name="s",
        num_cores=_NC, num_subcores=_NS)
    f = pl.kernel(
        _sc_body,
        out_type=jax.ShapeDtypeStruct((_B,), jnp.float32),
        mesh=mesh,
        scratch_types=[
            pltpu.VMEM((_V2PAD,), jnp.float32),
            pltpu.VMEM((_SUB * _F1,), jnp.int32),
            pltpu.VMEM((_SUB * _F2,), jnp.int32),
            pltpu.VMEM((_F1 * _SUB,), jnp.int32),
            pltpu.VMEM((_F1 * _SUB,), jnp.float32),
            pltpu.VMEM((_SPW,), jnp.float32),
            pltpu.VMEM((_L,), jnp.float32),
            pltpu.SemaphoreType.DMA,
        ],
        compiler_params=pltpu.CompilerParams(needs_layout_passes=False),
    )
    return f(x1f, x2f, w1, w2, bias16)


def kernel(x1, x2, W1, W2, bias1):
    x1f = x1.reshape(_B * _F1)
    x2f = x2.reshape(_B * _F2)
    w1 = W1.reshape(_V1)
    w2 = jnp.pad(W2.reshape(_V2), (0, _V2PAD - _V2))
    bias16 = jnp.broadcast_to(bias1.astype(jnp.float32), (_L,))
    return _run(x1f, x2f, w1, w2, bias16)
